# baseline (speedup 1.0000x reference)
import jax
import jax.numpy as jnp
from jax import lax
from jax.experimental import pallas as pl
from jax.experimental.pallas import tpu as pltpu
from jax.experimental.pallas import tpu_sc as plsc

N = 50000
NR = 4
MAX_DELAY = 5
E = 3200000
B = 12
DT = 1.0

C = 400
EPT = E // 16
WINS = EPT // C
HN = N // 2
HS = 25600
ACC_ROWS = NR * HS
CH = ACC_ROWS // 16


def _sc_body(z_rows, src_h, tgt_h, w_h, zeros_h, out_h,
             acc, sb0, sb1, tb0, tb1, wb0, wb1, rb0, rb1, ib0, ib1,
             es0, es1, et0, et1, ew0, ew1, gs0, gs1, ss0, ss1):
  ci = lax.axis_index("c")
  si = lax.axis_index("s")
  base_n = ci * HN
  e0 = si * EPT

  sbufs = (sb0, sb1)
  tbufs = (tb0, tb1)
  wbufs = (wb0, wb1)
  rbufs = (rb0, rb1)
  ibufs = (ib0, ib1)
  esems = (es0, es1)
  etsems = (et0, et1)
  ewsems = (ew0, ew1)
  gsems = (gs0, gs1)
  ssems = (ss0, ss1)

  pltpu.sync_copy(zeros_h, rb0)
  zstart = si * CH
  for kk in range(CH // C):
    pltpu.sync_copy(rb0, acc.at[pl.ds(zstart + kk * C, C)])
  plsc.subcore_barrier()

  def fire_edges(win, p):
    off = pl.multiple_of(e0 + win * C, 8)
    pltpu.async_copy(src_h.at[pl.ds(off, C)], sbufs[p], esems[p])
    pltpu.async_copy(tgt_h.at[pl.ds(off, C)], tbufs[p], etsems[p])
    pltpu.async_copy(w_h.at[pl.ds(off, C)], wbufs[p], ewsems[p])

  fire_edges(0, 0)
  fire_edges(1, 1)
  pltpu.make_async_copy(src_h.at[pl.ds(0, C)], sb0, es0).wait()
  pltpu.async_copy(z_rows.at[sb0], rb0, gs0)

  iota16 = lax.iota(jnp.int32, 16)
  gd = lax.GatherDimensionNumbers(offset_dims=(), collapsed_slice_dims=(0,),
                                  start_index_map=(0,))

  def splat(vec, l):
    return lax.gather(vec, jnp.full((16, 1), l, jnp.int32), gd, (1,),
                      mode=lax.GatherScatterMode.PROMISE_IN_BOUNDS)

  def win_body(g, carry):
    for p in range(2):
      pn = 1 - p

      if p == 0:
        pltpu.make_async_copy(src_h.at[pl.ds(0, C)], sbufs[pn],
                              esems[pn]).wait()
      else:
        @pl.when(g <= WINS // 2 - 2)
        def _():
          pltpu.make_async_copy(src_h.at[pl.ds(0, C)], sbufs[pn],
                                esems[pn]).wait()

      pltpu.make_async_copy(tgt_h.at[pl.ds(0, C)], tbufs[p], etsems[p]).wait()

      def idx_body(j, c):
        tv = tbufs[p][pl.ds(j * 16, 16)]
        nn = lax.shift_right_logical(tv, 2)
        rr = lax.bitwise_and(tv, 3)
        local = rr * HS + (nn - base_n)
        inr = (nn >= base_n) & (nn < base_n + HN)
        idx = jnp.where(inr, local, HN + (iota16 & 15))
        ibufs[p][pl.ds(j * 16, 16)] = idx
        return c

      lax.fori_loop(0, C // 16, idx_body, 0)

      if p == 0:
        @pl.when(g >= 1)
        def _():
          pltpu.make_async_copy(rbufs[pn], acc.at[ibufs[pn]],
                                ssems[pn]).wait()

        pltpu.async_copy(z_rows.at[sbufs[pn]], rbufs[pn], gsems[pn])
      else:
        pltpu.make_async_copy(rbufs[pn], acc.at[ibufs[pn]], ssems[pn]).wait()

        @pl.when(g <= WINS // 2 - 2)
        def _():
          pltpu.async_copy(z_rows.at[sbufs[pn]], rbufs[pn], gsems[pn])

      pltpu.make_async_copy(z_rows.at[sbufs[p]], rbufs[p], gsems[p]).wait()
      pltpu.make_async_copy(w_h.at[pl.ds(0, C)], wbufs[p], ewsems[p]).wait()

      def mul_body(j, c):
        wv = wbufs[p][pl.ds(j * 16, 16)]
        for l in range(16):
          e = j * 16 + l
          ws = splat(wv, l)
          rbufs[p][e] = rbufs[p][e] * ws
        return c

      lax.fori_loop(0, C // 16, mul_body, 0, unroll=2)

      @pl.when(g <= WINS // 2 - 2)
      def _():
        fire_edges(g * 2 + p + 2, p)

      pltpu.async_copy(rbufs[p], acc.at[ibufs[p]], ssems[p], add=True)
    return carry

  lax.fori_loop(0, WINS // 2, win_body, 0)

  pltpu.make_async_copy(rbufs[1], acc.at[ibufs[1]], ssems[1]).wait()
  plsc.subcore_barrier()

  W0 = 1568
  W1 = HN - 15 * W0
  for rr in range(NR):
    asrc = rr * HS + si * W0
    adst = ci * HN + si * W0

    @pl.when(si < 15)
    def _():
      pltpu.sync_copy(acc.at[pl.ds(asrc, W0)],
                      out_h.at[rr, pl.ds(adst, W0)])

    @pl.when(si == 15)
    def _():
      pltpu.sync_copy(acc.at[pl.ds(asrc, W1)],
                      out_h.at[rr, pl.ds(adst, W1)])


ZW = 2000
ZCHUNKS = (N * MAX_DELAY) // ZW
ZPER = -(-ZCHUNKS // 32)


def _zt_body(z_hbm, zeros_h, out_h, tin, tout, isem, osem):
  wid = lax.axis_index("c") * 16 + lax.axis_index("s")
  iota16 = lax.iota(jnp.int32, 16)
  pltpu.sync_copy(zeros_h, tin.at[pl.ds(B, 16 - B)])

  def chunk_body(c, carry):
    chunk = wid * ZPER + c

    @pl.when(chunk < ZCHUNKS)
    def _():
      off = pl.multiple_of(chunk * ZW, 8)
      for b in range(B):
        pltpu.async_copy(z_hbm.at[b, pl.ds(off, ZW)], tin.at[b], isem)
      for b in range(B):
        pltpu.make_async_copy(z_hbm.at[b, pl.ds(off, ZW)], tin.at[b],
                              isem).wait()

      def tr(j, c2):
        row = plsc.load_gather(tin, [iota16, jnp.broadcast_to(j, (16,))])
        tout[j] = row
        return c2

      lax.fori_loop(0, ZW, tr, 0)
      pltpu.async_copy(tout, out_h.at[pl.ds(off, ZW)], osem)
      pltpu.make_async_copy(tout, out_h.at[pl.ds(off, ZW)], osem).wait()

    return carry

  lax.fori_loop(0, ZPER, chunk_body, 0)


def _sc_transpose(z_buf, zeros_z):
  mesh = plsc.VectorSubcoreMesh(core_axis_name="c", subcore_axis_name="s")
  f = pl.kernel(
      _zt_body,
      out_type=jax.ShapeDtypeStruct((N * MAX_DELAY, 16), jnp.float32),
      mesh=mesh,
      compiler_params=pltpu.CompilerParams(use_tc_tiling_on_sc=False,
                                           needs_layout_passes=False),
      scratch_types=[
          pltpu.VMEM((16, ZW), jnp.float32),
          pltpu.VMEM((ZW, 16), jnp.float32),
          pltpu.SemaphoreType.DMA, pltpu.SemaphoreType.DMA,
      ],
  )
  return f(z_buf, zeros_z)


def _sc_sparse(z_rows, src, tgt, w, zeros_c):
  mesh = plsc.VectorSubcoreMesh(core_axis_name="c", subcore_axis_name="s")
  f = pl.kernel(
      _sc_body,
      out_type=jax.ShapeDtypeStruct((NR, N, 16), jnp.float32),
      mesh=mesh,
      compiler_params=pltpu.CompilerParams(use_tc_tiling_on_sc=False),
      scratch_types=[
          pltpu.VMEM_SHARED((ACC_ROWS, 16), jnp.float32),
          pltpu.VMEM((C,), jnp.int32), pltpu.VMEM((C,), jnp.int32),
          pltpu.VMEM((C,), jnp.int32), pltpu.VMEM((C,), jnp.int32),
          pltpu.VMEM((C,), jnp.float32), pltpu.VMEM((C,), jnp.float32),
          pltpu.VMEM((C, 16), jnp.float32), pltpu.VMEM((C, 16), jnp.float32),
          pltpu.VMEM((C,), jnp.int32), pltpu.VMEM((C,), jnp.int32),
          pltpu.SemaphoreType.DMA, pltpu.SemaphoreType.DMA,
          pltpu.SemaphoreType.DMA, pltpu.SemaphoreType.DMA,
          pltpu.SemaphoreType.DMA, pltpu.SemaphoreType.DMA,
          pltpu.SemaphoreType.DMA, pltpu.SemaphoreType.DMA,
          pltpu.SemaphoreType.DMA, pltpu.SemaphoreType.DMA,
      ],
  )
  return f(z_rows, src, tgt, w, zeros_c)


TN = 4096


def _dense_body(rec4, prt, pct, sdt, pit, v, r, asc1, asc2, ext,
                pz, vth, el, vrst, dec, cf, pg, tref, a0, a1, k0, k1,
                nz_o, nv_o, nr_o, na1_o, na2_o, npr_o, npc_o):
  pr = prt[...]
  pc = pct[...]
  syn = sdt[...]
  pit_ = pit[...]
  for rr in range(NR):
    rec_r = jnp.transpose(rec4[rr])[:B]
    npr_o[rr] = syn[rr] * pr[rr] + rec_r * pit_[rr]
    npc_o[rr] = pc[rr] * syn[rr] + DT * syn[rr] * pr[rr]

  prev_z = pz[...]
  new_r = jnp.maximum(r[...] + prev_z * tref[...] - DT, 0.0)
  nr_o[...] = new_r

  na1_o[...] = jnp.exp(-DT * k0[...]) * asc1[...] + prev_z * a0[...]
  na2_o[...] = jnp.exp(-DT * k1[...]) * asc2[...] + prev_z * a1[...]

  input_current = ((pc[0] + pc[1]) + pc[2]) + pc[3] + ext[...]
  c1 = input_current + asc1[...] + asc2[...] + pg[...] * el[...]
  new_v0 = dec[...] * v[...] + cf[...] * c1
  v_sc = (new_v0 - vth[...]) / (vth[...] - el[...])
  spike = (v_sc > 0.0).astype(jnp.float32)
  new_z = jnp.where(new_r > 0.0, 0.0, spike)
  nz_o[...] = new_z
  nv_o[...] = new_v0 - new_z * (vth[...] - vrst[...])


def _dense_call(rec4, prt, pct, sdt, pit, v, r, asc1, asc2, ext,
                pz, vth, el, vrst, dec, cf, pg, tref, a0, a1, k0, k1):
  g = pl.cdiv(N, TN)
  s4 = pl.BlockSpec((NR, TN, 16), lambda i: (0, i, 0))
  s3 = pl.BlockSpec((NR, B, TN), lambda i: (0, 0, i))
  s3p = pl.BlockSpec((NR, 1, TN), lambda i: (0, 0, i))
  s2 = pl.BlockSpec((B, TN), lambda i: (0, i))
  s2p = pl.BlockSpec((1, TN), lambda i: (0, i))
  f2 = jax.ShapeDtypeStruct((B, N), jnp.float32)
  f3 = jax.ShapeDtypeStruct((NR, B, N), jnp.float32)
  return pl.pallas_call(
      _dense_body,
      grid=(g,),
      in_specs=[s4, s3, s3, s3p, s3p, s2, s2, s2, s2, s2, s2,
                s2p, s2p, s2p, s2p, s2p, s2p, s2p, s2p, s2p, s2p, s2p],
      out_specs=[s2, s2, s2, s2, s2, s3, s3],
      out_shape=[f2, f2, f2, f2, f2, f3, f3],
  )(rec4, prt, pct, sdt, pit, v, r, asc1, asc2, ext, pz,
    vth, el, vrst, dec, cf, pg, tref, a0, a1, k0, k1)


def kernel(z_buf, v, r, asc_1, asc_2, psc_rise, psc, external_current,
           rec_indices, rec_weights, v_th, e_l, v_reset, decay,
           current_factor, param_g, t_ref, asc_amps, k, syn_decay,
           psc_initial):
  src = rec_indices[:, 1]
  tgt = rec_indices[:, 0]
  zeros_z = jnp.zeros((16 - B, ZW), jnp.float32)
  z_rows = _sc_transpose(z_buf, zeros_z)
  zeros_c = jnp.zeros((C, 16), jnp.float32)
  i_rows = _sc_sparse(z_rows, src, tgt, rec_weights, zeros_c)

  prt = psc_rise.reshape(B, N, NR).transpose(2, 0, 1)
  pct = psc.reshape(B, N, NR).transpose(2, 0, 1)
  sdt = syn_decay.T.reshape(NR, 1, N)
  pit = psc_initial.T.reshape(NR, 1, N)
  pz = z_buf[:, :N]
  row = lambda x: x.reshape(1, N)

  new_z, new_v, new_r, na1, na2, nprt, npct = _dense_call(
      i_rows, prt, pct, sdt, pit, v, r, asc_1, asc_2,
      external_current, pz,
      row(v_th), row(e_l), row(v_reset), row(decay), row(current_factor),
      row(param_g), row(t_ref),
      row(asc_amps[:, 0]), row(asc_amps[:, 1]), row(k[:, 0]), row(k[:, 1]))

  new_psc_rise = nprt.transpose(1, 2, 0).reshape(B, N * NR)
  new_psc = npct.transpose(1, 2, 0).reshape(B, N * NR)
  new_z_buf = jnp.concatenate([new_z, z_buf[:, :N * (MAX_DELAY - 1)]], axis=1)
  return (new_z, new_v, new_r, na1, na2, new_psc_rise, new_psc, new_z_buf)


# speedup vs baseline: 1.0045x; 1.0045x over previous
import jax
import jax.numpy as jnp
from jax import lax
from jax.experimental import pallas as pl
from jax.experimental.pallas import tpu as pltpu
from jax.experimental.pallas import tpu_sc as plsc

N = 50000
NR = 4
MAX_DELAY = 5
E = 3200000
B = 12
DT = 1.0

C = 400
EPT = E // 16
WINS = EPT // C
HN = N // 2
HS = 25600
ACC_ROWS = NR * HS
CH = ACC_ROWS // 16


def _sc_body(z_rows, src_h, tgt_h, w_h, zeros_h, out_h,
             acc, sb0, sb1, tb0, tb1, wb0, wb1, rb0, rb1, ib0, ib1,
             es0, es1, et0, et1, ew0, ew1, gs0, gs1, ss0, ss1):
  ci = lax.axis_index("c")
  si = lax.axis_index("s")
  base_n = ci * HN
  e0 = si * EPT

  sbufs = (sb0, sb1)
  tbufs = (tb0, tb1)
  wbufs = (wb0, wb1)
  rbufs = (rb0, rb1)
  ibufs = (ib0, ib1)
  esems = (es0, es1)
  etsems = (et0, et1)
  ewsems = (ew0, ew1)
  gsems = (gs0, gs1)
  ssems = (ss0, ss1)

  pltpu.sync_copy(zeros_h, rb0)
  zstart = si * CH
  for kk in range(CH // C):
    pltpu.sync_copy(rb0, acc.at[pl.ds(zstart + kk * C, C)])
  plsc.subcore_barrier()

  def fire_edges(win, p):
    off = pl.multiple_of(e0 + win * C, 8)
    pltpu.async_copy(src_h.at[pl.ds(off, C)], sbufs[p], esems[p])
    pltpu.async_copy(tgt_h.at[pl.ds(off, C)], tbufs[p], etsems[p])
    pltpu.async_copy(w_h.at[pl.ds(off, C)], wbufs[p], ewsems[p])

  fire_edges(0, 0)
  fire_edges(1, 1)
  pltpu.make_async_copy(src_h.at[pl.ds(0, C)], sb0, es0).wait()
  pltpu.async_copy(z_rows.at[sb0], rb0, gs0)

  iota16 = lax.iota(jnp.int32, 16)
  gd = lax.GatherDimensionNumbers(offset_dims=(), collapsed_slice_dims=(0,),
                                  start_index_map=(0,))

  def splat(vec, l):
    return lax.gather(vec, jnp.full((16, 1), l, jnp.int32), gd, (1,),
                      mode=lax.GatherScatterMode.PROMISE_IN_BOUNDS)

  def win_body(g, carry):
    for p in range(2):
      pn = 1 - p

      if p == 0:
        pltpu.make_async_copy(src_h.at[pl.ds(0, C)], sbufs[pn],
                              esems[pn]).wait()
      else:
        @pl.when(g <= WINS // 2 - 2)
        def _():
          pltpu.make_async_copy(src_h.at[pl.ds(0, C)], sbufs[pn],
                                esems[pn]).wait()

      pltpu.make_async_copy(tgt_h.at[pl.ds(0, C)], tbufs[p], etsems[p]).wait()

      def idx_body(j, c):
        tv = tbufs[p][pl.ds(j * 16, 16)]
        nn = lax.shift_right_logical(tv, 2)
        rr = lax.bitwise_and(tv, 3)
        local = rr * HS + (nn - base_n)
        inr = (nn >= base_n) & (nn < base_n + HN)
        idx = jnp.where(inr, local, HN + (iota16 & 15))
        ibufs[p][pl.ds(j * 16, 16)] = idx
        return c

      lax.fori_loop(0, C // 16, idx_body, 0)

      if p == 0:
        @pl.when(g >= 1)
        def _():
          pltpu.make_async_copy(rbufs[pn], acc.at[ibufs[pn]],
                                ssems[pn]).wait()

        pltpu.async_copy(z_rows.at[sbufs[pn]], rbufs[pn], gsems[pn])
      else:
        pltpu.make_async_copy(rbufs[pn], acc.at[ibufs[pn]], ssems[pn]).wait()

        @pl.when(g <= WINS // 2 - 2)
        def _():
          pltpu.async_copy(z_rows.at[sbufs[pn]], rbufs[pn], gsems[pn])

      pltpu.make_async_copy(z_rows.at[sbufs[p]], rbufs[p], gsems[p]).wait()
      pltpu.make_async_copy(w_h.at[pl.ds(0, C)], wbufs[p], ewsems[p]).wait()

      def mul_body(j, c):
        wv = wbufs[p][pl.ds(j * 16, 16)]
        for l in range(16):
          e = j * 16 + l
          ws = splat(wv, l)
          rbufs[p][e] = rbufs[p][e] * ws
        return c

      lax.fori_loop(0, C // 16, mul_body, 0)

      @pl.when(g <= WINS // 2 - 2)
      def _():
        fire_edges(g * 2 + p + 2, p)

      pltpu.async_copy(rbufs[p], acc.at[ibufs[p]], ssems[p], add=True)
    return carry

  lax.fori_loop(0, WINS // 2, win_body, 0)

  pltpu.make_async_copy(rbufs[1], acc.at[ibufs[1]], ssems[1]).wait()
  plsc.subcore_barrier()

  W0 = 1568
  W1 = HN - 15 * W0
  for rr in range(NR):
    asrc = rr * HS + si * W0
    adst = ci * HN + si * W0

    @pl.when(si < 15)
    def _():
      pltpu.sync_copy(acc.at[pl.ds(asrc, W0)],
                      out_h.at[rr, pl.ds(adst, W0)])

    @pl.when(si == 15)
    def _():
      pltpu.sync_copy(acc.at[pl.ds(asrc, W1)],
                      out_h.at[rr, pl.ds(adst, W1)])


ZW = 2000
ZCHUNKS = (N * MAX_DELAY) // ZW
ZPER = -(-ZCHUNKS // 32)


def _zt_body(z_hbm, zeros_h, out_h, tin, tout, isem, osem):
  wid = lax.axis_index("c") * 16 + lax.axis_index("s")
  iota16 = lax.iota(jnp.int32, 16)
  pltpu.sync_copy(zeros_h, tin.at[pl.ds(B, 16 - B)])

  def chunk_body(c, carry):
    chunk = wid * ZPER + c

    @pl.when(chunk < ZCHUNKS)
    def _():
      off = pl.multiple_of(chunk * ZW, 8)
      for b in range(B):
        pltpu.async_copy(z_hbm.at[b, pl.ds(off, ZW)], tin.at[b], isem)
      for b in range(B):
        pltpu.make_async_copy(z_hbm.at[b, pl.ds(off, ZW)], tin.at[b],
                              isem).wait()

      def tr(j, c2):
        row = plsc.load_gather(tin, [iota16, jnp.broadcast_to(j, (16,))])
        tout[j] = row
        return c2

      lax.fori_loop(0, ZW, tr, 0)
      pltpu.async_copy(tout, out_h.at[pl.ds(off, ZW)], osem)
      pltpu.make_async_copy(tout, out_h.at[pl.ds(off, ZW)], osem).wait()

    return carry

  lax.fori_loop(0, ZPER, chunk_body, 0)


def _sc_transpose(z_buf, zeros_z):
  mesh = plsc.VectorSubcoreMesh(core_axis_name="c", subcore_axis_name="s")
  f = pl.kernel(
      _zt_body,
      out_type=jax.ShapeDtypeStruct((N * MAX_DELAY, 16), jnp.float32),
      mesh=mesh,
      compiler_params=pltpu.CompilerParams(use_tc_tiling_on_sc=False,
                                           needs_layout_passes=False),
      scratch_types=[
          pltpu.VMEM((16, ZW), jnp.float32),
          pltpu.VMEM((ZW, 16), jnp.float32),
          pltpu.SemaphoreType.DMA, pltpu.SemaphoreType.DMA,
      ],
  )
  return f(z_buf, zeros_z)


def _sc_sparse(z_rows, src, tgt, w, zeros_c):
  mesh = plsc.VectorSubcoreMesh(core_axis_name="c", subcore_axis_name="s")
  f = pl.kernel(
      _sc_body,
      out_type=jax.ShapeDtypeStruct((NR, N, 16), jnp.float32),
      mesh=mesh,
      compiler_params=pltpu.CompilerParams(use_tc_tiling_on_sc=False),
      scratch_types=[
          pltpu.VMEM_SHARED((ACC_ROWS, 16), jnp.float32),
          pltpu.VMEM((C,), jnp.int32), pltpu.VMEM((C,), jnp.int32),
          pltpu.VMEM((C,), jnp.int32), pltpu.VMEM((C,), jnp.int32),
          pltpu.VMEM((C,), jnp.float32), pltpu.VMEM((C,), jnp.float32),
          pltpu.VMEM((C, 16), jnp.float32), pltpu.VMEM((C, 16), jnp.float32),
          pltpu.VMEM((C,), jnp.int32), pltpu.VMEM((C,), jnp.int32),
          pltpu.SemaphoreType.DMA, pltpu.SemaphoreType.DMA,
          pltpu.SemaphoreType.DMA, pltpu.SemaphoreType.DMA,
          pltpu.SemaphoreType.DMA, pltpu.SemaphoreType.DMA,
          pltpu.SemaphoreType.DMA, pltpu.SemaphoreType.DMA,
          pltpu.SemaphoreType.DMA, pltpu.SemaphoreType.DMA,
      ],
  )
  return f(z_rows, src, tgt, w, zeros_c)


TN = 4096


def _dense_body(rec4, prt, pct, sdt, pit, v, r, asc1, asc2, ext,
                pz, vth, el, vrst, dec, cf, pg, tref, a0, a1, k0, k1,
                nz_o, nv_o, nr_o, na1_o, na2_o, npr_o, npc_o):
  pr = prt[...]
  pc = pct[...]
  syn = sdt[...]
  pit_ = pit[...]
  for rr in range(NR):
    rec_r = jnp.transpose(rec4[rr])[:B]
    npr_o[rr] = syn[rr] * pr[rr] + rec_r * pit_[rr]
    npc_o[rr] = pc[rr] * syn[rr] + DT * syn[rr] * pr[rr]

  prev_z = pz[...]
  new_r = jnp.maximum(r[...] + prev_z * tref[...] - DT, 0.0)
  nr_o[...] = new_r

  na1_o[...] = jnp.exp(-DT * k0[...]) * asc1[...] + prev_z * a0[...]
  na2_o[...] = jnp.exp(-DT * k1[...]) * asc2[...] + prev_z * a1[...]

  input_current = ((pc[0] + pc[1]) + pc[2]) + pc[3] + ext[...]
  c1 = input_current + asc1[...] + asc2[...] + pg[...] * el[...]
  new_v0 = dec[...] * v[...] + cf[...] * c1
  v_sc = (new_v0 - vth[...]) / (vth[...] - el[...])
  spike = (v_sc > 0.0).astype(jnp.float32)
  new_z = jnp.where(new_r > 0.0, 0.0, spike)
  nz_o[...] = new_z
  nv_o[...] = new_v0 - new_z * (vth[...] - vrst[...])


def _dense_call(rec4, prt, pct, sdt, pit, v, r, asc1, asc2, ext,
                pz, vth, el, vrst, dec, cf, pg, tref, a0, a1, k0, k1):
  g = pl.cdiv(N, TN)
  s4 = pl.BlockSpec((NR, TN, 16), lambda i: (0, i, 0))
  s3 = pl.BlockSpec((NR, B, TN), lambda i: (0, 0, i))
  s3p = pl.BlockSpec((NR, 1, TN), lambda i: (0, 0, i))
  s2 = pl.BlockSpec((B, TN), lambda i: (0, i))
  s2p = pl.BlockSpec((1, TN), lambda i: (0, i))
  f2 = jax.ShapeDtypeStruct((B, N), jnp.float32)
  f3 = jax.ShapeDtypeStruct((NR, B, N), jnp.float32)
  return pl.pallas_call(
      _dense_body,
      grid=(g,),
      in_specs=[s4, s3, s3, s3p, s3p, s2, s2, s2, s2, s2, s2,
                s2p, s2p, s2p, s2p, s2p, s2p, s2p, s2p, s2p, s2p, s2p],
      out_specs=[s2, s2, s2, s2, s2, s3, s3],
      out_shape=[f2, f2, f2, f2, f2, f3, f3],
  )(rec4, prt, pct, sdt, pit, v, r, asc1, asc2, ext, pz,
    vth, el, vrst, dec, cf, pg, tref, a0, a1, k0, k1)


def kernel(z_buf, v, r, asc_1, asc_2, psc_rise, psc, external_current,
           rec_indices, rec_weights, v_th, e_l, v_reset, decay,
           current_factor, param_g, t_ref, asc_amps, k, syn_decay,
           psc_initial):
  src = rec_indices[:, 1]
  tgt = rec_indices[:, 0]
  zeros_z = jnp.zeros((16 - B, ZW), jnp.float32)
  z_rows = _sc_transpose(z_buf, zeros_z)
  zeros_c = jnp.zeros((C, 16), jnp.float32)
  i_rows = _sc_sparse(z_rows, src, tgt, rec_weights, zeros_c)

  prt = psc_rise.reshape(B, N, NR).transpose(2, 0, 1)
  pct = psc.reshape(B, N, NR).transpose(2, 0, 1)
  sdt = syn_decay.T.reshape(NR, 1, N)
  pit = psc_initial.T.reshape(NR, 1, N)
  pz = z_buf[:, :N]
  row = lambda x: x.reshape(1, N)

  new_z, new_v, new_r, na1, na2, nprt, npct = _dense_call(
      i_rows, prt, pct, sdt, pit, v, r, asc_1, asc_2,
      external_current, pz,
      row(v_th), row(e_l), row(v_reset), row(decay), row(current_factor),
      row(param_g), row(t_ref),
      row(asc_amps[:, 0]), row(asc_amps[:, 1]), row(k[:, 0]), row(k[:, 1]))

  new_psc_rise = nprt.transpose(1, 2, 0).reshape(B, N * NR)
  new_psc = npct.transpose(1, 2, 0).reshape(B, N * NR)
  new_z_buf = jnp.concatenate([new_z, z_buf[:, :N * (MAX_DELAY - 1)]], axis=1)
  return (new_z, new_v, new_r, na1, na2, new_psc_rise, new_psc, new_z_buf)
